# Initial kernel scaffold; baseline (speedup 1.0000x reference)
import jax
import jax.numpy as jnp
from jax.experimental import pallas as pl


def kernel(inputs, targets, superpixels, spmasks):
    raise NotImplementedError("write your pallas kernel here")


# speedup vs baseline: 3.1958x; 3.1958x over previous
import functools

import jax
import jax.numpy as jnp
from jax import lax
from jax.experimental import pallas as pl
from jax.experimental.pallas import tpu as pltpu
from jax.experimental.pallas import tpu_sc as plsc

_NUM_SP = 2048
_TEMP = 1.0
_EPS = 1e-08
_C = 19

_NC = 2
_NS = 16
_NW = _NC * _NS
_L = 16

_R = 2
_CHUNK = 8192
_ACCROWS = _NUM_SP + 1
_ACCSIZE = _L * _ACCROWS


def _softmax_idx_body(x_ref, sp_ref, sm_ref, prob_ref, idx_ref):
    x = x_ref[0] * (1.0 / _TEMP)
    m = jnp.max(x, axis=0, keepdims=True)
    e = jnp.exp(x - m)
    s = jnp.sum(e, axis=0, keepdims=True)
    prob_ref[0] = e / s
    idx_ref[0, 0] = jnp.where(sm_ref[0, 0] != 0, sp_ref[0, 0], _NUM_SP)


def _softmax_call(x3, sp3, sm3, pb):
    n, c, p = x3.shape
    nblk = p // pb
    grid = (n, nblk)
    return pl.pallas_call(
        _softmax_idx_body,
        grid=grid,
        in_specs=[
            pl.BlockSpec((1, c, pb), lambda i, j: (i, 0, j)),
            pl.BlockSpec((1, 1, pb), lambda i, j, _nb=nblk: (i * _nb + j, 0, 0)),
            pl.BlockSpec((1, 1, pb), lambda i, j, _nb=nblk: (i * _nb + j, 0, 0)),
        ],
        out_specs=[
            pl.BlockSpec((1, c, pb), lambda i, j: (i, 0, j)),
            pl.BlockSpec((1, 1, pb), lambda i, j, _nb=nblk: (i * _nb + j, 0, 0)),
        ],
        out_shape=[
            jax.ShapeDtypeStruct((n, c, p), jnp.float32),
            jax.ShapeDtypeStruct((n * nblk, 1, pb), jnp.int32),
        ],
    )(x3, sp3, sm3)


def _scatter_max_call(probs, idxm):
    n, c, p = probs.shape
    half = p // _R
    nchunk = half // _CHUNK
    units = n * c * _R
    rounds = (units + _NW - 1) // _NW
    mesh = plsc.VectorSubcoreMesh(core_axis_name="c", subcore_axis_name="s")

    @functools.partial(
        pl.kernel,
        mesh=mesh,
        compiler_params=pltpu.CompilerParams(needs_layout_passes=False),
        out_type=jax.ShapeDtypeStruct((n, c, _R, _NUM_SP), jnp.float32),
        scratch_types=[
            pltpu.VMEM((_CHUNK,), jnp.float32),
            pltpu.VMEM((_CHUNK,), jnp.int32),
            pltpu.VMEM((_ACCSIZE,), jnp.float32),
            pltpu.VMEM((_NUM_SP,), jnp.float32),
        ],
    )
    def _sc(probs_hbm, idx_hbm, out_hbm, pbuf, ibuf, acc, obuf):
        wid = lax.axis_index("s") * _NC + lax.axis_index("c")
        lane = lax.iota(jnp.int32, _L) * _ACCROWS
        zero16 = jnp.zeros((_L,), jnp.float32)
        for k in range(rounds):
            u = wid + k * _NW

            @pl.when(u < units)
            def _():
                img = u // (c * _R)
                rem = u % (c * _R)
                ch = rem // _R
                r = rem % _R
                p0 = r * half

                def zbody(j, carry):
                    acc[pl.ds(j * _L, _L)] = zero16
                    return carry

                lax.fori_loop(0, _ACCROWS, zbody, 0)

                def cbody(t, carry):
                    off = p0 + t * _CHUNK
                    pltpu.sync_copy(probs_hbm.at[img, ch, pl.ds(off, _CHUNK)], pbuf)
                    pltpu.sync_copy(idx_hbm.at[img, pl.ds(off, _CHUNK)], ibuf)

                    def vbody(v, inner):
                        sl = pl.ds(v * _L, _L)
                        addr = ibuf[sl] + lane
                        old = plsc.load_gather(acc, [addr])
                        plsc.store_scatter(acc, [addr], jnp.maximum(old, pbuf[sl]))
                        return inner

                    lax.fori_loop(0, _CHUNK // _L, vbody, 0)
                    return carry

                lax.fori_loop(0, nchunk, cbody, 0)

                def rbody(b, carry):
                    base = b * _L
                    mx = acc[pl.ds(base, _L)]
                    for l in range(1, _L):
                        mx = jnp.maximum(mx, acc[pl.ds(l * _ACCROWS + base, _L)])
                    obuf[pl.ds(base, _L)] = mx
                    return carry

                lax.fori_loop(0, _NUM_SP // _L, rbody, 0)
                pltpu.sync_copy(obuf, out_hbm.at[img, ch, r])

    return _sc(probs, idxm)


def _ce_body(part_ref, trg_ref, out_ref):
    total = jnp.float32(0.0)
    cnt = jnp.float32(0.0)
    for i in range(part_ref.shape[0]):
        pm = part_ref[i, :, 0]
        for r in range(1, _R):
            pm = jnp.maximum(pm, part_ref[i, :, r])
        trg = trg_ref[i][:, :_C]
        rowmask = jnp.any(trg != 0, axis=1)
        top = pm.T * trg * rowmask[:, None].astype(jnp.float32)
        nz = top > 0
        cnt = cnt + jnp.sum(nz.astype(jnp.float32))
        total = total + jnp.sum(jnp.where(nz, -jnp.log(top + _EPS), 0.0))
    out_ref[0, 0] = total / (cnt + 1.0)


def _ce_call(partials, targets):
    return pl.pallas_call(
        _ce_body,
        out_specs=pl.BlockSpec(memory_space=pltpu.SMEM),
        out_shape=jax.ShapeDtypeStruct((1, 1), jnp.float32),
    )(partials, targets)


def kernel(inputs, targets, superpixels, spmasks):
    n, c, h, w = inputs.shape
    p = h * w
    pb = 32768
    nblk = p // pb
    x3 = inputs.reshape(n, c, p)
    sp3 = superpixels.astype(jnp.int32).reshape(n * nblk, 1, pb)
    sm3 = spmasks.astype(jnp.int32).reshape(n * nblk, 1, pb)
    probs, idxm = _softmax_call(x3, sp3, sm3, pb)
    partials = _scatter_max_call(probs, idxm.reshape(n, p))
    loss = _ce_call(partials, targets)
    return loss[0, 0]


# speedup vs baseline: 3.9844x; 1.2468x over previous
import functools

import jax
import jax.numpy as jnp
from jax import lax
from jax.experimental import pallas as pl
from jax.experimental.pallas import tpu as pltpu
from jax.experimental.pallas import tpu_sc as plsc

_NUM_SP = 2048
_TEMP = 1.0
_EPS = 1e-08
_C = 19

_NC = 2
_NS = 16
_NW = _NC * _NS
_L = 16

_R = 2
_CHUNK = 8192
_ACCROWS = _NUM_SP + 1
_ACCSIZE = _L * _ACCROWS


def _softmax_idx_body(x_ref, sp_ref, sm_ref, prob_ref, idx_ref):
    x = x_ref[0] * (1.0 / _TEMP)
    m = jnp.max(x, axis=0, keepdims=True)
    e = jnp.exp(x - m)
    s = jnp.sum(e, axis=0, keepdims=True)
    prob_ref[0] = e / s
    idx_ref[0, 0] = jnp.where(sm_ref[0, 0] != 0, sp_ref[0, 0], _NUM_SP)


def _softmax_call(x3, sp3, sm3, pb):
    n, c, p = x3.shape
    nblk = p // pb
    grid = (n, nblk)
    return pl.pallas_call(
        _softmax_idx_body,
        grid=grid,
        in_specs=[
            pl.BlockSpec((1, c, pb), lambda i, j: (i, 0, j)),
            pl.BlockSpec((1, 1, pb), lambda i, j, _nb=nblk: (i * _nb + j, 0, 0)),
            pl.BlockSpec((1, 1, pb), lambda i, j, _nb=nblk: (i * _nb + j, 0, 0)),
        ],
        out_specs=[
            pl.BlockSpec((1, c, pb), lambda i, j: (i, 0, j)),
            pl.BlockSpec((1, 1, pb), lambda i, j, _nb=nblk: (i * _nb + j, 0, 0)),
        ],
        out_shape=[
            jax.ShapeDtypeStruct((n, c, p), jnp.float32),
            jax.ShapeDtypeStruct((n * nblk, 1, pb), jnp.int32),
        ],
    )(x3, sp3, sm3)


def _scatter_max_call(probs, idxm):
    n, c, p = probs.shape
    half = p // _R
    nchunk = half // _CHUNK
    units = n * c * _R
    rounds = (units + _NW - 1) // _NW
    mesh = plsc.VectorSubcoreMesh(core_axis_name="c", subcore_axis_name="s")

    @functools.partial(
        pl.kernel,
        mesh=mesh,
        compiler_params=pltpu.CompilerParams(needs_layout_passes=False),
        out_type=jax.ShapeDtypeStruct((n, c, _R, _NUM_SP), jnp.float32),
        scratch_types=[
            pltpu.VMEM((2, _CHUNK), jnp.float32),
            pltpu.VMEM((2, _CHUNK), jnp.int32),
            pltpu.VMEM((2 * _ACCSIZE,), jnp.float32),
            pltpu.VMEM((_NUM_SP,), jnp.float32),
            pltpu.SemaphoreType.DMA,
            pltpu.SemaphoreType.DMA,
        ],
    )
    def _sc(probs_hbm, idx_hbm, out_hbm, pbuf, ibuf, acc, obuf, psem, isem):
        wid = lax.axis_index("s") * _NC + lax.axis_index("c")
        lane = lax.iota(jnp.int32, _L) * _ACCROWS
        zero16 = jnp.zeros((_L,), jnp.float32)
        for k in range(rounds):
            u = wid + k * _NW

            @pl.when(u < units)
            def _():
                img = u // (c * _R)
                rem = u % (c * _R)
                ch = rem // _R
                r = rem % _R
                p0 = r * half

                def zbody(j, carry):
                    acc[pl.ds(j * _L, _L)] = zero16
                    return carry

                lax.fori_loop(0, 2 * _ACCROWS, zbody, 0)

                def start_dma(t, buf):
                    off = p0 + t * _CHUNK
                    pltpu.async_copy(
                        probs_hbm.at[img, ch, pl.ds(off, _CHUNK)], pbuf.at[buf], psem
                    )
                    pltpu.async_copy(idx_hbm.at[img, pl.ds(off, _CHUNK)], ibuf.at[buf], isem)

                def wait_dma(buf):
                    pltpu.make_async_copy(
                        probs_hbm.at[img, ch, pl.ds(0, _CHUNK)], pbuf.at[buf], psem
                    ).wait()
                    pltpu.make_async_copy(
                        idx_hbm.at[img, pl.ds(0, _CHUNK)], ibuf.at[buf], isem
                    ).wait()

                start_dma(0, 0)

                def cbody(t, carry):
                    cur = lax.rem(t, 2)
                    wait_dma(cur)

                    @pl.when(t + 1 < nchunk)
                    def _():
                        start_dma(t + 1, 1 - cur)

                    def vbody(v, inner):
                        a0 = ibuf[cur, pl.ds(2 * v * _L, _L)] + lane
                        old0 = plsc.load_gather(acc, [a0])
                        a1 = ibuf[cur, pl.ds((2 * v + 1) * _L, _L)] + lane + _ACCSIZE
                        old1 = plsc.load_gather(acc, [a1])
                        plsc.store_scatter(
                            acc, [a0], jnp.maximum(old0, pbuf[cur, pl.ds(2 * v * _L, _L)])
                        )
                        plsc.store_scatter(
                            acc,
                            [a1],
                            jnp.maximum(old1, pbuf[cur, pl.ds((2 * v + 1) * _L, _L)]),
                        )
                        return inner

                    lax.fori_loop(0, _CHUNK // (2 * _L), vbody, 0)
                    return carry

                lax.fori_loop(0, nchunk, cbody, 0)

                def rbody(b, carry):
                    base = b * _L
                    mx = acc[pl.ds(base, _L)]
                    for l in range(1, 2 * _L):
                        mx = jnp.maximum(mx, acc[pl.ds(l * _ACCROWS + base, _L)])
                    obuf[pl.ds(base, _L)] = mx
                    return carry

                lax.fori_loop(0, _NUM_SP // _L, rbody, 0)
                pltpu.sync_copy(obuf, out_hbm.at[img, ch, r])

    return _sc(probs, idxm)


def _ce_body(part_ref, trg_ref, out_ref):
    total = jnp.float32(0.0)
    cnt = jnp.float32(0.0)
    for i in range(part_ref.shape[0]):
        pm = part_ref[i, :, 0]
        for r in range(1, _R):
            pm = jnp.maximum(pm, part_ref[i, :, r])
        trg = trg_ref[i][:, :_C]
        rowmask = jnp.any(trg != 0, axis=1)
        top = pm.T * trg * rowmask[:, None].astype(jnp.float32)
        nz = top > 0
        cnt = cnt + jnp.sum(nz.astype(jnp.float32))
        total = total + jnp.sum(jnp.where(nz, -jnp.log(top + _EPS), 0.0))
    out_ref[0, 0] = total / (cnt + 1.0)


def _ce_call(partials, targets):
    return pl.pallas_call(
        _ce_body,
        out_specs=pl.BlockSpec(memory_space=pltpu.SMEM),
        out_shape=jax.ShapeDtypeStruct((1, 1), jnp.float32),
    )(partials, targets)


def kernel(inputs, targets, superpixels, spmasks):
    n, c, h, w = inputs.shape
    p = h * w
    pb = 32768
    nblk = p // pb
    x3 = inputs.reshape(n, c, p)
    sp3 = superpixels.astype(jnp.int32).reshape(n * nblk, 1, pb)
    sm3 = spmasks.astype(jnp.int32).reshape(n * nblk, 1, pb)
    probs, idxm = _softmax_call(x3, sp3, sm3, pb)
    partials = _scatter_max_call(probs, idxm.reshape(n, p))
    loss = _ce_call(partials, targets)
    return loss[0, 0]


# speedup vs baseline: 4.2544x; 1.0678x over previous
import functools

import jax
import jax.numpy as jnp
from jax import lax
from jax.experimental import pallas as pl
from jax.experimental.pallas import tpu as pltpu
from jax.experimental.pallas import tpu_sc as plsc

_NUM_SP = 2048
_TEMP = 1.0
_EPS = 1e-08
_C = 19

_NC = 2
_NS = 16
_NW = _NC * _NS
_L = 16

_R = 2
_CHUNK = 8192
_ACCROWS = _NUM_SP + 1
_ACCSIZE = _L * _ACCROWS


def _softmax_idx_body(x_ref, sp_ref, sm_ref, prob_ref, idx_ref):
    x = x_ref[0] * (1.0 / _TEMP)
    m = jnp.max(x, axis=0, keepdims=True)
    e = jnp.exp(x - m)
    s = jnp.sum(e, axis=0, keepdims=True)
    prob_ref[0] = e / s
    idx_ref[0, 0] = jnp.where(sm_ref[0, 0] != 0, sp_ref[0, 0], _NUM_SP)


def _softmax_call(x3, sp3, sm3, pb):
    n, c, p = x3.shape
    nblk = p // pb
    grid = (n, nblk)
    return pl.pallas_call(
        _softmax_idx_body,
        grid=grid,
        in_specs=[
            pl.BlockSpec((1, c, pb), lambda i, j: (i, 0, j)),
            pl.BlockSpec((1, 1, pb), lambda i, j, _nb=nblk: (i * _nb + j, 0, 0)),
            pl.BlockSpec((1, 1, pb), lambda i, j, _nb=nblk: (i * _nb + j, 0, 0)),
        ],
        out_specs=[
            pl.BlockSpec((1, c, pb), lambda i, j: (i, 0, j)),
            pl.BlockSpec((1, 1, pb), lambda i, j, _nb=nblk: (i * _nb + j, 0, 0)),
        ],
        out_shape=[
            jax.ShapeDtypeStruct((n, c, p), jnp.float32),
            jax.ShapeDtypeStruct((n * nblk, 1, pb), jnp.int32),
        ],
    )(x3, sp3, sm3)


def _scatter_max_call(probs, idxm):
    n, c, p = probs.shape
    half = p // _R
    nchunk = half // _CHUNK
    units = n * c * _R
    rounds = (units + _NW - 1) // _NW
    mesh = plsc.VectorSubcoreMesh(core_axis_name="c", subcore_axis_name="s")

    @functools.partial(
        pl.kernel,
        mesh=mesh,
        compiler_params=pltpu.CompilerParams(needs_layout_passes=False),
        out_type=jax.ShapeDtypeStruct((n, c, _R, _NUM_SP), jnp.float32),
        scratch_types=[
            pltpu.VMEM((2, _CHUNK), jnp.float32),
            pltpu.VMEM((2, _CHUNK), jnp.int32),
            pltpu.VMEM((_ACCSIZE,), jnp.float32),
            pltpu.VMEM((_ACCSIZE,), jnp.float32),
            pltpu.VMEM((_NUM_SP,), jnp.float32),
            pltpu.SemaphoreType.DMA,
            pltpu.SemaphoreType.DMA,
        ],
    )
    def _sc(probs_hbm, idx_hbm, out_hbm, pbuf, ibuf, acc0, acc1, obuf, psem, isem):
        wid = lax.axis_index("s") * _NC + lax.axis_index("c")
        lane = lax.iota(jnp.int32, _L) * _ACCROWS
        zero16 = jnp.zeros((_L,), jnp.float32)
        for k in range(rounds):
            u = wid + k * _NW

            @pl.when(u < units)
            def _():
                img = u // (c * _R)
                rem = u % (c * _R)
                ch = rem // _R
                r = rem % _R
                p0 = r * half

                def zbody(j, carry):
                    acc0[pl.ds(j * _L, _L)] = zero16
                    acc1[pl.ds(j * _L, _L)] = zero16
                    return carry

                lax.fori_loop(0, _ACCROWS, zbody, 0)

                def start_dma(t, buf):
                    off = p0 + t * _CHUNK
                    pltpu.async_copy(
                        probs_hbm.at[img, ch, pl.ds(off, _CHUNK)], pbuf.at[buf], psem
                    )
                    pltpu.async_copy(idx_hbm.at[img, pl.ds(off, _CHUNK)], ibuf.at[buf], isem)

                def wait_dma(buf):
                    pltpu.make_async_copy(
                        probs_hbm.at[img, ch, pl.ds(0, _CHUNK)], pbuf.at[buf], psem
                    ).wait()
                    pltpu.make_async_copy(
                        idx_hbm.at[img, pl.ds(0, _CHUNK)], ibuf.at[buf], isem
                    ).wait()

                start_dma(0, 0)

                def cbody(t, carry):
                    cur = lax.rem(t, 2)
                    wait_dma(cur)

                    @pl.when(t + 1 < nchunk)
                    def _():
                        start_dma(t + 1, 1 - cur)

                    def vbody(v, inner):
                        a0 = ibuf[cur, pl.ds(2 * v * _L, _L)] + lane
                        old0 = plsc.load_gather(acc0, [a0])
                        a1 = ibuf[cur, pl.ds((2 * v + 1) * _L, _L)] + lane
                        old1 = plsc.load_gather(acc1, [a1])
                        plsc.store_scatter(
                            acc0, [a0], jnp.maximum(old0, pbuf[cur, pl.ds(2 * v * _L, _L)])
                        )
                        plsc.store_scatter(
                            acc1,
                            [a1],
                            jnp.maximum(old1, pbuf[cur, pl.ds((2 * v + 1) * _L, _L)]),
                        )
                        return inner

                    lax.fori_loop(0, _CHUNK // (2 * _L), vbody, 0)
                    return carry

                lax.fori_loop(0, nchunk, cbody, 0)

                def rbody(b, carry):
                    base = b * _L
                    mx = jnp.maximum(acc0[pl.ds(base, _L)], acc1[pl.ds(base, _L)])
                    for l in range(1, _L):
                        mx = jnp.maximum(mx, acc0[pl.ds(l * _ACCROWS + base, _L)])
                        mx = jnp.maximum(mx, acc1[pl.ds(l * _ACCROWS + base, _L)])
                    obuf[pl.ds(base, _L)] = mx
                    return carry

                lax.fori_loop(0, _NUM_SP // _L, rbody, 0)
                pltpu.sync_copy(obuf, out_hbm.at[img, ch, r])

    return _sc(probs, idxm)


def _ce_body(part_ref, trg_ref, out_ref):
    total = jnp.float32(0.0)
    cnt = jnp.float32(0.0)
    for i in range(part_ref.shape[0]):
        pm = part_ref[i, :, 0]
        for r in range(1, _R):
            pm = jnp.maximum(pm, part_ref[i, :, r])
        trg = trg_ref[i][:, :_C]
        rowmask = jnp.any(trg != 0, axis=1)
        top = pm.T * trg * rowmask[:, None].astype(jnp.float32)
        nz = top > 0
        cnt = cnt + jnp.sum(nz.astype(jnp.float32))
        total = total + jnp.sum(jnp.where(nz, -jnp.log(top + _EPS), 0.0))
    out_ref[0, 0] = total / (cnt + 1.0)


def _ce_call(partials, targets):
    return pl.pallas_call(
        _ce_body,
        out_specs=pl.BlockSpec(memory_space=pltpu.SMEM),
        out_shape=jax.ShapeDtypeStruct((1, 1), jnp.float32),
    )(partials, targets)


def kernel(inputs, targets, superpixels, spmasks):
    n, c, h, w = inputs.shape
    p = h * w
    pb = 32768
    nblk = p // pb
    x3 = inputs.reshape(n, c, p)
    sp3 = superpixels.astype(jnp.int32).reshape(n * nblk, 1, pb)
    sm3 = spmasks.astype(jnp.int32).reshape(n * nblk, 1, pb)
    probs, idxm = _softmax_call(x3, sp3, sm3, pb)
    partials = _scatter_max_call(probs, idxm.reshape(n, p))
    loss = _ce_call(partials, targets)
    return loss[0, 0]


# speedup vs baseline: 4.6585x; 1.0950x over previous
import functools

import jax
import jax.numpy as jnp
from jax import lax
from jax.experimental import pallas as pl
from jax.experimental.pallas import tpu as pltpu
from jax.experimental.pallas import tpu_sc as plsc

_NUM_SP = 2048
_TEMP = 1.0
_EPS = 1e-08
_C = 19

_NC = 2
_NS = 16
_NW = _NC * _NS
_L = 16

_R = 2
_CHUNK = 4096
_ACCROWS = _NUM_SP + 1
_ACCSIZE = _L * _ACCROWS


def _softmax_idx_body(x_ref, sp_ref, sm_ref, prob_ref, idx_ref):
    x = x_ref[0] * (1.0 / _TEMP)
    m = jnp.max(x, axis=0, keepdims=True)
    e = jnp.exp(x - m)
    s = jnp.sum(e, axis=0, keepdims=True)
    prob_ref[0] = e / s
    idx_ref[0, 0] = jnp.where(sm_ref[0, 0] != 0, sp_ref[0, 0], _NUM_SP)


def _softmax_call(x3, sp3, sm3, pb):
    n, c, p = x3.shape
    nblk = p // pb
    grid = (n, nblk)
    return pl.pallas_call(
        _softmax_idx_body,
        grid=grid,
        in_specs=[
            pl.BlockSpec((1, c, pb), lambda i, j: (i, 0, j)),
            pl.BlockSpec((1, 1, pb), lambda i, j, _nb=nblk: (i * _nb + j, 0, 0)),
            pl.BlockSpec((1, 1, pb), lambda i, j, _nb=nblk: (i * _nb + j, 0, 0)),
        ],
        out_specs=[
            pl.BlockSpec((1, c, pb), lambda i, j: (i, 0, j)),
            pl.BlockSpec((1, 1, pb), lambda i, j, _nb=nblk: (i * _nb + j, 0, 0)),
        ],
        out_shape=[
            jax.ShapeDtypeStruct((n, c, p), jnp.float32),
            jax.ShapeDtypeStruct((n * nblk, 1, pb), jnp.int32),
        ],
    )(x3, sp3, sm3)


def _scatter_max_call(probs, idxm):
    n, c, p = probs.shape
    half = p // _R
    nchunk = half // _CHUNK
    units = n * c * _R
    rounds = (units + _NW - 1) // _NW
    mesh = plsc.VectorSubcoreMesh(core_axis_name="c", subcore_axis_name="s")

    @functools.partial(
        pl.kernel,
        mesh=mesh,
        compiler_params=pltpu.CompilerParams(needs_layout_passes=False),
        out_type=jax.ShapeDtypeStruct((n, c, _R, _NUM_SP), jnp.float32),
        scratch_types=[
            pltpu.VMEM((2, _CHUNK), jnp.float32),
            pltpu.VMEM((2, _CHUNK), jnp.int32),
            pltpu.VMEM((_ACCSIZE,), jnp.float32),
            pltpu.VMEM((_ACCSIZE,), jnp.float32),
            pltpu.VMEM((_ACCSIZE,), jnp.float32),
            pltpu.VMEM((_NUM_SP,), jnp.float32),
            pltpu.SemaphoreType.DMA,
            pltpu.SemaphoreType.DMA,
        ],
    )
    def _sc(probs_hbm, idx_hbm, out_hbm, pbuf, ibuf, acc0, acc1, acc2, obuf, psem, isem):
        wid = lax.axis_index("s") * _NC + lax.axis_index("c")
        lane = lax.iota(jnp.int32, _L) * _ACCROWS
        zero16 = jnp.zeros((_L,), jnp.float32)
        for k in range(rounds):
            u = wid + k * _NW

            @pl.when(u < units)
            def _():
                img = u // (c * _R)
                rem = u % (c * _R)
                ch = rem // _R
                r = rem % _R
                p0 = r * half

                def zbody(j, carry):
                    acc0[pl.ds(j * _L, _L)] = zero16
                    acc1[pl.ds(j * _L, _L)] = zero16
                    acc2[pl.ds(j * _L, _L)] = zero16
                    return carry

                lax.fori_loop(0, _ACCROWS, zbody, 0)

                def start_dma(t, buf):
                    off = p0 + t * _CHUNK
                    pltpu.async_copy(
                        probs_hbm.at[img, ch, pl.ds(off, _CHUNK)], pbuf.at[buf], psem
                    )
                    pltpu.async_copy(idx_hbm.at[img, pl.ds(off, _CHUNK)], ibuf.at[buf], isem)

                def wait_dma(buf):
                    pltpu.make_async_copy(
                        probs_hbm.at[img, ch, pl.ds(0, _CHUNK)], pbuf.at[buf], psem
                    ).wait()
                    pltpu.make_async_copy(
                        idx_hbm.at[img, pl.ds(0, _CHUNK)], ibuf.at[buf], isem
                    ).wait()

                start_dma(0, 0)

                def cbody(t, carry):
                    cur = lax.rem(t, 2)
                    wait_dma(cur)

                    @pl.when(t + 1 < nchunk)
                    def _():
                        start_dma(t + 1, 1 - cur)

                    def vbody(v, inner):
                        a0 = ibuf[cur, pl.ds(3 * v * _L, _L)] + lane
                        old0 = plsc.load_gather(acc0, [a0])
                        a1 = ibuf[cur, pl.ds((3 * v + 1) * _L, _L)] + lane
                        old1 = plsc.load_gather(acc1, [a1])
                        a2 = ibuf[cur, pl.ds((3 * v + 2) * _L, _L)] + lane
                        old2 = plsc.load_gather(acc2, [a2])
                        plsc.store_scatter(
                            acc0, [a0], jnp.maximum(old0, pbuf[cur, pl.ds(3 * v * _L, _L)])
                        )
                        plsc.store_scatter(
                            acc1,
                            [a1],
                            jnp.maximum(old1, pbuf[cur, pl.ds((3 * v + 1) * _L, _L)]),
                        )
                        plsc.store_scatter(
                            acc2,
                            [a2],
                            jnp.maximum(old2, pbuf[cur, pl.ds((3 * v + 2) * _L, _L)]),
                        )
                        return inner

                    nfull = _CHUNK // _L // 3
                    lax.fori_loop(0, nfull, vbody, 0)
                    for w in range(3 * nfull, _CHUNK // _L):
                        aw = ibuf[cur, pl.ds(w * _L, _L)] + lane
                        oldw = plsc.load_gather(acc0, [aw])
                        plsc.store_scatter(
                            acc0, [aw], jnp.maximum(oldw, pbuf[cur, pl.ds(w * _L, _L)])
                        )
                    return carry

                lax.fori_loop(0, nchunk, cbody, 0)

                def rbody(b, carry):
                    base = b * _L
                    mx = jnp.maximum(
                        jnp.maximum(acc0[pl.ds(base, _L)], acc1[pl.ds(base, _L)]),
                        acc2[pl.ds(base, _L)],
                    )
                    for l in range(1, _L):
                        mx = jnp.maximum(mx, acc0[pl.ds(l * _ACCROWS + base, _L)])
                        mx = jnp.maximum(mx, acc1[pl.ds(l * _ACCROWS + base, _L)])
                        mx = jnp.maximum(mx, acc2[pl.ds(l * _ACCROWS + base, _L)])
                    obuf[pl.ds(base, _L)] = mx
                    return carry

                lax.fori_loop(0, _NUM_SP // _L, rbody, 0)
                pltpu.sync_copy(obuf, out_hbm.at[img, ch, r])

    return _sc(probs, idxm)


def _ce_body(part_ref, trg_ref, out_ref):
    total = jnp.float32(0.0)
    cnt = jnp.float32(0.0)
    for i in range(part_ref.shape[0]):
        pm = part_ref[i, :, 0]
        for r in range(1, _R):
            pm = jnp.maximum(pm, part_ref[i, :, r])
        trg = trg_ref[i][:, :_C]
        rowmask = jnp.any(trg != 0, axis=1)
        top = pm.T * trg * rowmask[:, None].astype(jnp.float32)
        nz = top > 0
        cnt = cnt + jnp.sum(nz.astype(jnp.float32))
        total = total + jnp.sum(jnp.where(nz, -jnp.log(top + _EPS), 0.0))
    out_ref[0, 0] = total / (cnt + 1.0)


def _ce_call(partials, targets):
    return pl.pallas_call(
        _ce_body,
        out_specs=pl.BlockSpec(memory_space=pltpu.SMEM),
        out_shape=jax.ShapeDtypeStruct((1, 1), jnp.float32),
    )(partials, targets)


def kernel(inputs, targets, superpixels, spmasks):
    n, c, h, w = inputs.shape
    p = h * w
    pb = 32768
    nblk = p // pb
    x3 = inputs.reshape(n, c, p)
    sp3 = superpixels.astype(jnp.int32).reshape(n * nblk, 1, pb)
    sm3 = spmasks.astype(jnp.int32).reshape(n * nblk, 1, pb)
    probs, idxm = _softmax_call(x3, sp3, sm3, pb)
    partials = _scatter_max_call(probs, idxm.reshape(n, p))
    loss = _ce_call(partials, targets)
    return loss[0, 0]
